# Initial kernel scaffold; baseline (speedup 1.0000x reference)
import functools

import jax
import jax.numpy as jnp
from jax import lax
from jax.experimental import pallas as pl
from jax.experimental.pallas import tpu as pltpu
from jax.experimental.pallas import tpu_sc as plsc

_LANES = 16
_VOCAB_PAD = 160
_EMBED = 32


def _fold_tables(E_r, E_d, W1):
  def body(er, ed, w1, o):
    w = w1[...]
    a = jnp.dot(er[...], w[0:_EMBED, :], preferred_element_type=jnp.float32)
    b = jnp.dot(ed[...], w[_EMBED:2 * _EMBED, :], preferred_element_type=jnp.float32)
    z = jnp.zeros((_VOCAB_PAD - a.shape[0], _EMBED), jnp.float32)
    o[...] = jnp.concatenate([a, z, b, z], axis=0)

  return pl.pallas_call(
      body,
      out_shape=jax.ShapeDtypeStruct((2 * _VOCAB_PAD, _EMBED), jnp.float32),
  )(E_r, E_d, W1)


@functools.lru_cache(maxsize=None)
def _build_sc(B, H):
  info = plsc.get_sparse_core_info()
  NC, NS = info.num_cores, info.num_subcores
  NW = NC * NS
  bw = B // NW
  G = bw // _LANES
  TABN = 2 * _VOCAB_PAD * _EMBED
  DBASE = _VOCAB_PAD * _EMBED
  NP = 32 * 16 + 32 + 16 + 16 + 1
  NPP = ((NP + 7) // 8) * 8

  mesh = plsc.VectorSubcoreMesh(core_axis_name="c", subcore_axis_name="s")

  @functools.partial(
      pl.kernel,
      mesh=mesh,
      out_type=jax.ShapeDtypeStruct((B,), jnp.float32),
      scratch_types=[
          pltpu.VMEM((bw * H,), jnp.int32),
          pltpu.VMEM((bw * H,), jnp.int32),
          pltpu.VMEM((TABN,), jnp.float32),
          pltpu.VMEM((NPP,), jnp.float32),
          pltpu.VMEM((bw,), jnp.float32),
      ],
  )
  def sck(tab_hbm, par_hbm, ir_hbm, id_hbm, out_hbm, ir_v, id_v, tab_v, par_v, out_v):
    wid = lax.axis_index("s") * NC + lax.axis_index("c")
    base = wid * bw
    pltpu.sync_copy(ir_hbm.at[pl.ds(base * H, bw * H)], ir_v)
    pltpu.sync_copy(id_hbm.at[pl.ds(base * H, bw * H)], id_v)
    pltpu.sync_copy(tab_hbm, tab_v)
    pltpu.sync_copy(par_hbm, par_v)

    iotaH = lax.iota(jnp.int32, _LANES) * H

    def group(g, carry):
      off = g * (_LANES * H)
      acc = [jnp.full((_LANES,), par_v[512 + k], jnp.float32) for k in range(_EMBED)]
      for idxv, cbase in ((ir_v, 0), (id_v, DBASE)):
        for h in range(H):
          pos = iotaH + (off + h)
          ids = plsc.load_gather(idxv, [pos])
          ids32 = ids * _EMBED
          for k in range(_EMBED):
            val = plsc.load_gather(tab_v, [ids32 + (cbase + k)])
            acc[k] = acc[k] + val
      h1 = [jnp.maximum(a, 0.0) for a in acc]
      outv = jnp.full((_LANES,), par_v[576], jnp.float32)
      for j in range(16):
        a = jnp.full((_LANES,), par_v[544 + j], jnp.float32)
        for k in range(_EMBED):
          a = a + h1[k] * par_v[k * 16 + j]
        a = jnp.maximum(a, 0.0)
        outv = outv + a * par_v[560 + j]
      outv = jnp.maximum(outv, 0.0)
      out_v[pl.ds(g * _LANES, _LANES)] = outv
      return carry

    lax.fori_loop(0, G, group, 0)
    pltpu.sync_copy(out_v, out_hbm.at[pl.ds(base, bw)])

  return sck


def kernel(radiant_heros, dire_heros, E_r, E_d, W1, b1, W2, b2, W3, b3):
  B, H = radiant_heros.shape
  table = _fold_tables(E_r, E_d, W1).reshape(-1)
  pad = (-(32 * 16 + 32 + 16 + 16 + 1)) % 8
  params = jnp.concatenate([
      W2.reshape(-1), b1, b2, W3.reshape(-1), b3,
      jnp.zeros((pad,), jnp.float32),
  ])
  sck = _build_sc(B, H)
  out = sck(table, params, radiant_heros.reshape(-1), dire_heros.reshape(-1))
  return out.reshape(B, 1)


# speedup vs baseline: 5.6941x; 5.6941x over previous
import functools

import jax
import jax.numpy as jnp
from jax import lax
from jax.experimental import pallas as pl
from jax.experimental.pallas import tpu as pltpu
from jax.experimental.pallas import tpu_sc as plsc

_LANES = 16
_VOCAB_PAD = 160
_EMBED = 32


def _fold_tables(E_r, E_d, W1):
  def body(er, ed, w1, o):
    w = w1[...]
    a = jnp.dot(er[...], w[0:_EMBED, :], preferred_element_type=jnp.float32,
                precision=jax.lax.Precision.HIGHEST)
    b = jnp.dot(ed[...], w[_EMBED:2 * _EMBED, :], preferred_element_type=jnp.float32,
                precision=jax.lax.Precision.HIGHEST)
    z = jnp.zeros((_VOCAB_PAD - a.shape[0], _EMBED), jnp.float32)
    o[...] = jnp.concatenate([a, z, b, z], axis=0)

  return pl.pallas_call(
      body,
      out_shape=jax.ShapeDtypeStruct((2 * _VOCAB_PAD, _EMBED), jnp.float32),
  )(E_r, E_d, W1)


@functools.lru_cache(maxsize=None)
def _build_sc(B, H):
  info = plsc.get_sparse_core_info()
  NC, NS = info.num_cores, info.num_subcores
  NW = NC * NS
  bw = B // NW
  G = bw // _LANES
  TABN = 2 * _VOCAB_PAD * _EMBED
  DBASE = _VOCAB_PAD * _EMBED
  NP = 32 * 16 + 32 + 16 + 16 + 1
  NPP = ((NP + 15) // 16) * 16

  mesh = plsc.VectorSubcoreMesh(core_axis_name="c", subcore_axis_name="s")

  @functools.partial(
      pl.kernel,
      mesh=mesh,
      out_type=jax.ShapeDtypeStruct((B,), jnp.float32),
      compiler_params=pltpu.CompilerParams(needs_layout_passes=False),
      scratch_types=[
          pltpu.VMEM((bw * H,), jnp.int32),
          pltpu.VMEM((bw * H,), jnp.int32),
          pltpu.VMEM((TABN,), jnp.float32),
          pltpu.VMEM((NPP,), jnp.float32),
          pltpu.VMEM((bw,), jnp.float32),
      ],
  )
  def sck(tab_hbm, par_hbm, ir_hbm, id_hbm, out_hbm, ir_v, id_v, tab_v, par_v, out_v):
    wid = lax.axis_index("s") * NC + lax.axis_index("c")
    base = wid * bw
    pltpu.sync_copy(ir_hbm.at[pl.ds(base * H, bw * H)], ir_v)
    pltpu.sync_copy(id_hbm.at[pl.ds(base * H, bw * H)], id_v)
    pltpu.sync_copy(tab_hbm, tab_v)
    pltpu.sync_copy(par_hbm, par_v)

    iotaH = lax.iota(jnp.int32, _LANES) * H

    def group(g, carry):
      off = g * (_LANES * H)
      b1a = par_v[pl.ds(512, 16)]
      b1b = par_v[pl.ds(528, 16)]
      acc = [jnp.full((_LANES,), (b1a if k < 16 else b1b)[k % 16], jnp.float32)
             for k in range(_EMBED)]
      for idxv, cbase in ((ir_v, 0), (id_v, DBASE)):
        for h in range(H):
          pos = iotaH + (off + h)
          ids = plsc.load_gather(idxv, [pos])
          ids32 = ids * _EMBED
          for k in range(_EMBED):
            val = plsc.load_gather(tab_v, [ids32 + (cbase + k)])
            acc[k] = acc[k] + val
      h1 = [jnp.maximum(a, 0.0) for a in acc]
      b2v = par_v[pl.ds(544, 16)]
      w3v = par_v[pl.ds(560, 16)]
      b3v = par_v[pl.ds(576, 16)]
      a = [jnp.full((_LANES,), b2v[j], jnp.float32) for j in range(16)]
      for k in range(_EMBED):
        w = par_v[pl.ds(k * 16, 16)]
        for j in range(16):
          a[j] = a[j] + h1[k] * w[j]
      outv = jnp.full((_LANES,), b3v[0], jnp.float32)
      for j in range(16):
        outv = outv + jnp.maximum(a[j], 0.0) * w3v[j]
      outv = jnp.maximum(outv, 0.0)
      out_v[pl.ds(g * _LANES, _LANES)] = outv
      return carry

    lax.fori_loop(0, G, group, 0)
    pltpu.sync_copy(out_v, out_hbm.at[pl.ds(base, bw)])

  return sck


def kernel(radiant_heros, dire_heros, E_r, E_d, W1, b1, W2, b2, W3, b3):
  B, H = radiant_heros.shape
  table = _fold_tables(E_r, E_d, W1).reshape(-1)
  pad = (-(32 * 16 + 32 + 16 + 16 + 1)) % 16
  params = jnp.concatenate([
      W2.reshape(-1), b1, b2, W3.reshape(-1), b3,
      jnp.zeros((pad,), jnp.float32),
  ])
  sck = _build_sc(B, H)
  out = sck(table, params, radiant_heros.reshape(-1), dire_heros.reshape(-1))
  return out.reshape(B, 1)
